# baseline (speedup 1.0000x reference)
import functools

import jax
import jax.numpy as jnp
from jax import lax
from jax.experimental import pallas as pl
from jax.experimental.pallas import tpu as pltpu
from jax.experimental.pallas import tpu_sc as plsc

N = 10000
E = 320000
D = 128
H = 128

NC = 2
NS = 16
NW = NC * NS
EPW = E // NW
C = 40
KB = 8
NBLK = 32
EPWP = NBLK * KB * C
NPAD = 10240
RPT = NPAD // NS

EPT = E // NS
DC = 80
DNCH = EPT // DC


def _edge_pass_body(y_hbm, src_hbm, dst_hbm, out_hbm,
                    src_v, dst_v, rows0_v, rows1_v, rows2_v, acc_sh,
                    gsem0, gsem1, gsem2, ssem0, ssem1, ssem2):
    c = lax.axis_index("c")
    s = lax.axis_index("s")
    wid = s * NC + c
    rows = (rows0_v, rows1_v, rows2_v)
    gsems = (gsem0, gsem1, gsem2)
    ssems = (ssem0, ssem1, ssem2)

    def _zfill(i, _):
        for j in range(H // 16):
            rows0_v[i, pl.ds(j * 16, 16)] = jnp.zeros((16,), jnp.float32)
        return 0
    lax.fori_loop(0, C, _zfill, 0)
    for k in range(RPT // C):
        pltpu.sync_copy(rows0_v, acc_sh.at[pl.ds(s * RPT + k * C, C)])

    plsc.subcore_barrier()

    def _block(b, _):
        pltpu.sync_copy(src_hbm.at[wid, b], src_v)
        pltpu.sync_copy(dst_hbm.at[wid, b], dst_v)
        gh = [None] * KB
        sh = [None] * KB
        gh[0] = pltpu.async_copy(y_hbm.at[src_v.at[0]], rows[0], gsems[0])
        gh[1] = pltpu.async_copy(y_hbm.at[src_v.at[1]], rows[1], gsems[1])
        for j in range(KB):
            k = j % 3
            gh[j].wait()
            sh[j] = pltpu.async_copy(
                rows[k], acc_sh.at[dst_v.at[j]], ssems[k], add=True)
            if j + 2 < KB:
                if j >= 1:
                    sh[j - 1].wait()
                gh[j + 2] = pltpu.async_copy(
                    y_hbm.at[src_v.at[j + 2]], rows[(j + 2) % 3],
                    gsems[(j + 2) % 3])
        sh[KB - 3].wait()
        sh[KB - 2].wait()
        sh[KB - 1].wait()
        return 0
    lax.fori_loop(0, NBLK, _block, 0)

    plsc.subcore_barrier()

    pltpu.sync_copy(acc_sh.at[pl.ds(s * RPT, RPT)],
                    out_hbm.at[c, pl.ds(s * RPT, RPT)])


def _deg_pass_body(dst_hbm, out_hbm,
                   dst_v, ones_v, acc_sh):
    c = lax.axis_index("c")
    s = lax.axis_index("s")

    def _fill(val):
        def body(i, _):
            ones_v[i, pl.ds(0, 16)] = jnp.full((16,), val, jnp.float32)
            return 0
        lax.fori_loop(0, DC, body, 0)
    _fill(0.0)
    for k in range(RPT // DC):
        pltpu.sync_copy(ones_v, acc_sh.at[pl.ds(s * RPT + k * DC, DC)])
    _fill(1.0)

    pltpu.sync_copy(dst_hbm.at[c, s], dst_v)

    plsc.subcore_barrier()

    def _chunk(i, _):
        pltpu.sync_copy(ones_v, acc_sh.at[dst_v.at[i]], add=True)
        return 0
    lax.fori_loop(0, DNCH, _chunk, 0)

    plsc.subcore_barrier()

    pltpu.sync_copy(acc_sh.at[pl.ds(s * RPT, RPT)],
                    out_hbm.at[c, pl.ds(s * RPT, RPT)])


@functools.lru_cache(maxsize=None)
def _sc_kernels():
    mesh = plsc.VectorSubcoreMesh(core_axis_name="c", subcore_axis_name="s")
    edge = pl.kernel(
        _edge_pass_body,
        mesh=mesh,
        out_type=jax.ShapeDtypeStruct((NC, NPAD, H), jnp.float32),
        scratch_types=[
            pltpu.VMEM((KB, C), jnp.int32),
            pltpu.VMEM((KB, C), jnp.int32),
            pltpu.VMEM((C, H), jnp.float32),
            pltpu.VMEM((C, H), jnp.float32),
            pltpu.VMEM((C, H), jnp.float32),
            pltpu.VMEM_SHARED((NPAD, H), jnp.float32),
            pltpu.SemaphoreType.DMA,
            pltpu.SemaphoreType.DMA,
            pltpu.SemaphoreType.DMA,
            pltpu.SemaphoreType.DMA,
            pltpu.SemaphoreType.DMA,
            pltpu.SemaphoreType.DMA,
        ],
    )
    deg = pl.kernel(
        _deg_pass_body,
        mesh=mesh,
        out_type=jax.ShapeDtypeStruct((NC, NPAD, 16), jnp.float32),
        scratch_types=[
            pltpu.VMEM((DNCH, DC), jnp.int32),
            pltpu.VMEM((DC, 16), jnp.float32),
            pltpu.VMEM_SHARED((NPAD, 16), jnp.float32),
        ],
    )
    return edge, deg


_B = 1000


def _dinv_blk(degt):
    return lax.rsqrt(1.0 + degt[0])[:, 0:1]


def _elu(a):
    return jnp.where(a > 0.0, a, jnp.exp(a) - 1.0)


def _prep_body(x_ref, w_ref, degt_ref, out_ref):
    dinv = _dinv_blk(degt_ref[...])
    xw = jnp.dot(x_ref[...], w_ref[...], preferred_element_type=jnp.float32)
    out_ref[...] = xw * dinv


def _mid_body(p_ref, y_ref, degt_p_ref, b_ref, w_ref, degt_n_ref,
              *out_refs, keep_act):
    dinv_p = _dinv_blk(degt_p_ref[...])
    a = (p_ref[0] + p_ref[1] + y_ref[...]) * dinv_p + b_ref[...]
    act = _elu(a)
    dinv_n = _dinv_blk(degt_n_ref[...])
    yn = jnp.dot(act, w_ref[...], preferred_element_type=jnp.float32) * dinv_n
    if keep_act:
        out_refs[0][...] = act
        out_refs[1][...] = yn
    else:
        out_refs[0][...] = yn


def _fin_body(p_ref, y_ref, degt_ref, b_ref, out_ref):
    dinv = _dinv_blk(degt_ref[...])
    a = (p_ref[0] + p_ref[1] + y_ref[...]) * dinv + b_ref[...]
    out_ref[...] = jnp.maximum(a, 0.0)


_spec_rows = pl.BlockSpec((_B, H), lambda i: (i, 0))
_spec_pair = pl.BlockSpec((2, _B, H), lambda i: (0, i, 0))
_spec_w = pl.BlockSpec((D, H), lambda i: (0, 0))
_spec_b = pl.BlockSpec((1, H), lambda i: (0, 0))
_GRID = (N // _B,)
_rows_t = jax.ShapeDtypeStruct((N, H), jnp.float32)


def _spec_deg(l):
    return pl.BlockSpec((1, _B, 16), lambda i: (l, i, 0))


def _prep(x, w, degt, l):
    return pl.pallas_call(
        _prep_body, grid=_GRID,
        in_specs=[_spec_rows, _spec_w, _spec_deg(l)],
        out_specs=_spec_rows, out_shape=_rows_t,
    )(x, w, degt)


def _mid(p, y, degt, lp, b, w, ln, keep_act):
    out_specs = (_spec_rows, _spec_rows) if keep_act else _spec_rows
    out_shape = (_rows_t, _rows_t) if keep_act else _rows_t
    return pl.pallas_call(
        functools.partial(_mid_body, keep_act=keep_act), grid=_GRID,
        in_specs=[_spec_pair, _spec_rows, _spec_deg(lp), _spec_b, _spec_w,
                  _spec_deg(ln)],
        out_specs=out_specs, out_shape=out_shape,
    )(p, y, degt, b.reshape(1, H), w, degt)


def _fin(p, y, degt, l, b):
    return pl.pallas_call(
        _fin_body, grid=_GRID,
        in_specs=[_spec_pair, _spec_rows, _spec_deg(l), _spec_b],
        out_specs=_spec_rows, out_shape=_rows_t,
    )(p, y, degt, b.reshape(1, H))


_NPADE = EPWP - EPW


def _pad_edges(idx, is_dst):
    per = idx.reshape(NW, EPW)
    k = jnp.arange(NW * _NPADE, dtype=jnp.int32).reshape(NW, _NPADE)
    if is_dst:
        pad = N + (k % (NPAD - N))
    else:
        pad = (k * 41) % N
    return jnp.concatenate([per, pad], axis=1).reshape(NW, NBLK, KB, C)


@jax.jit
def kernel(x, edge_indices, W1, b1, W2, b2, W3, b3, W4, b4):
    src1 = _pad_edges(edge_indices[0, 0], is_dst=False)
    dst1 = _pad_edges(edge_indices[0, 1], is_dst=True)
    src2 = _pad_edges(edge_indices[1, 0], is_dst=False)
    dst2 = _pad_edges(edge_indices[1, 1], is_dst=True)
    dstd = jnp.stack([edge_indices[0, 1].reshape(NS, DNCH, DC),
                      edge_indices[1, 1].reshape(NS, DNCH, DC)])

    _edge_pass, _deg_pass = _sc_kernels()
    dp = _deg_pass(dstd)

    y1 = _prep(x, W1, dp, 0)
    p1 = _edge_pass(y1, src1, dst1)
    y2 = _mid(p1, y1, dp, 0, b1, W2, 1, keep_act=False)
    p2 = _edge_pass(y2, src2, dst2)
    z, y3 = _mid(p2, y2, dp, 1, b2, W3, 0, keep_act=True)
    p3 = _edge_pass(y3, src1, dst1)
    y4 = _mid(p3, y3, dp, 0, b3, W4, 1, keep_act=False)
    p4 = _edge_pass(y4, src2, dst2)
    x_rec = _fin(p4, y4, dp, 1, b4)
    return (z, x_rec)


# speedup vs baseline: 1.2175x; 1.2175x over previous
import functools

import jax
import jax.numpy as jnp
from jax import lax
from jax.experimental import pallas as pl
from jax.experimental.pallas import tpu as pltpu
from jax.experimental.pallas import tpu_sc as plsc

N = 10000
E = 320000
D = 128
H = 128

NC = 2
NS = 16
NW = NC * NS
EPW = E // NW
C = 64
KB = 16
NBLK = 10
EPWP = NBLK * KB * C
NPAD = 10240
RPT = NPAD // NS

EPT = E // NS
DC = 80
DNCH = EPT // DC


def _edge_pass_body(y_hbm, src_hbm, dst_hbm, out_hbm,
                    src_v, dst_v, rows0_v, rows1_v, acc_sh,
                    gsem0, gsem1):
    c = lax.axis_index("c")
    s = lax.axis_index("s")
    wid = s * NC + c
    rows = (rows0_v, rows1_v)
    gsems = (gsem0, gsem1)

    def _zfill(i, _):
        for j in range(H // 16):
            rows0_v[i, pl.ds(j * 16, 16)] = jnp.zeros((16,), jnp.float32)
        return 0
    lax.fori_loop(0, C, _zfill, 0)
    for k in range(RPT // C):
        pltpu.sync_copy(rows0_v, acc_sh.at[pl.ds(s * RPT + k * C, C)])

    plsc.subcore_barrier()

    def _block(b, _):
        pltpu.sync_copy(src_hbm.at[wid, b], src_v)
        pltpu.sync_copy(dst_hbm.at[wid, b], dst_v)
        gh = [None] * KB
        gh[0] = pltpu.async_copy(y_hbm.at[src_v.at[0]], rows[0], gsems[0])
        for j in range(KB):
            if j + 1 < KB:
                gh[j + 1] = pltpu.async_copy(
                    y_hbm.at[src_v.at[j + 1]], rows[(j + 1) % 2],
                    gsems[(j + 1) % 2])
            gh[j].wait()
            pltpu.sync_copy(rows[j % 2], acc_sh.at[dst_v.at[j]], add=True)
        return 0
    lax.fori_loop(0, NBLK, _block, 0)

    plsc.subcore_barrier()

    pltpu.sync_copy(acc_sh.at[pl.ds(s * RPT, RPT)],
                    out_hbm.at[c, pl.ds(s * RPT, RPT)])


def _deg_pass_body(dst_hbm, out_hbm,
                   dst_v, ones_v, acc_sh):
    c = lax.axis_index("c")
    s = lax.axis_index("s")

    def _fill(val):
        def body(i, _):
            ones_v[i, pl.ds(0, 16)] = jnp.full((16,), val, jnp.float32)
            return 0
        lax.fori_loop(0, DC, body, 0)
    _fill(0.0)
    for k in range(RPT // DC):
        pltpu.sync_copy(ones_v, acc_sh.at[pl.ds(s * RPT + k * DC, DC)])
    _fill(1.0)

    pltpu.sync_copy(dst_hbm.at[c, s], dst_v)

    plsc.subcore_barrier()

    def _chunk(i, _):
        pltpu.sync_copy(ones_v, acc_sh.at[dst_v.at[i]], add=True)
        return 0
    lax.fori_loop(0, DNCH, _chunk, 0)

    plsc.subcore_barrier()

    pltpu.sync_copy(acc_sh.at[pl.ds(s * RPT, RPT)],
                    out_hbm.at[c, pl.ds(s * RPT, RPT)])


@functools.lru_cache(maxsize=None)
def _sc_kernels():
    mesh = plsc.VectorSubcoreMesh(core_axis_name="c", subcore_axis_name="s")
    edge = pl.kernel(
        _edge_pass_body,
        mesh=mesh,
        out_type=jax.ShapeDtypeStruct((NC, NPAD, H), jnp.float32),
        scratch_types=[
            pltpu.VMEM((KB, C), jnp.int32),
            pltpu.VMEM((KB, C), jnp.int32),
            pltpu.VMEM((C, H), jnp.float32),
            pltpu.VMEM((C, H), jnp.float32),
            pltpu.VMEM_SHARED((NPAD, H), jnp.float32),
            pltpu.SemaphoreType.DMA,
            pltpu.SemaphoreType.DMA,
        ],
    )
    deg = pl.kernel(
        _deg_pass_body,
        mesh=mesh,
        out_type=jax.ShapeDtypeStruct((NC, NPAD, 16), jnp.float32),
        scratch_types=[
            pltpu.VMEM((DNCH, DC), jnp.int32),
            pltpu.VMEM((DC, 16), jnp.float32),
            pltpu.VMEM_SHARED((NPAD, 16), jnp.float32),
        ],
    )
    return edge, deg


_B = 1000


def _dinv_blk(degt):
    return lax.rsqrt(1.0 + degt[0])[:, 0:1]


def _elu(a):
    return jnp.where(a > 0.0, a, jnp.exp(a) - 1.0)


def _prep_body(x_ref, w_ref, degt_ref, out_ref):
    dinv = _dinv_blk(degt_ref[...])
    xw = jnp.dot(x_ref[...], w_ref[...], preferred_element_type=jnp.float32)
    out_ref[...] = xw * dinv


def _mid_body(p_ref, y_ref, degt_p_ref, b_ref, w_ref, degt_n_ref,
              *out_refs, keep_act):
    dinv_p = _dinv_blk(degt_p_ref[...])
    a = (p_ref[0] + p_ref[1] + y_ref[...]) * dinv_p + b_ref[...]
    act = _elu(a)
    dinv_n = _dinv_blk(degt_n_ref[...])
    yn = jnp.dot(act, w_ref[...], preferred_element_type=jnp.float32) * dinv_n
    if keep_act:
        out_refs[0][...] = act
        out_refs[1][...] = yn
    else:
        out_refs[0][...] = yn


def _fin_body(p_ref, y_ref, degt_ref, b_ref, out_ref):
    dinv = _dinv_blk(degt_ref[...])
    a = (p_ref[0] + p_ref[1] + y_ref[...]) * dinv + b_ref[...]
    out_ref[...] = jnp.maximum(a, 0.0)


_spec_rows = pl.BlockSpec((_B, H), lambda i: (i, 0))
_spec_pair = pl.BlockSpec((2, _B, H), lambda i: (0, i, 0))
_spec_w = pl.BlockSpec((D, H), lambda i: (0, 0))
_spec_b = pl.BlockSpec((1, H), lambda i: (0, 0))
_GRID = (N // _B,)
_rows_t = jax.ShapeDtypeStruct((N, H), jnp.float32)


def _spec_deg(l):
    return pl.BlockSpec((1, _B, 16), lambda i: (l, i, 0))


def _prep(x, w, degt, l):
    return pl.pallas_call(
        _prep_body, grid=_GRID,
        in_specs=[_spec_rows, _spec_w, _spec_deg(l)],
        out_specs=_spec_rows, out_shape=_rows_t,
    )(x, w, degt)


def _mid(p, y, degt, lp, b, w, ln, keep_act):
    out_specs = (_spec_rows, _spec_rows) if keep_act else _spec_rows
    out_shape = (_rows_t, _rows_t) if keep_act else _rows_t
    return pl.pallas_call(
        functools.partial(_mid_body, keep_act=keep_act), grid=_GRID,
        in_specs=[_spec_pair, _spec_rows, _spec_deg(lp), _spec_b, _spec_w,
                  _spec_deg(ln)],
        out_specs=out_specs, out_shape=out_shape,
    )(p, y, degt, b.reshape(1, H), w, degt)


def _fin(p, y, degt, l, b):
    return pl.pallas_call(
        _fin_body, grid=_GRID,
        in_specs=[_spec_pair, _spec_rows, _spec_deg(l), _spec_b],
        out_specs=_spec_rows, out_shape=_rows_t,
    )(p, y, degt, b.reshape(1, H))


_NPADE = EPWP - EPW


def _pad_edges(idx, is_dst):
    per = idx.reshape(NW, EPW)
    k = jnp.arange(NW * _NPADE, dtype=jnp.int32).reshape(NW, _NPADE)
    if is_dst:
        pad = N + (k % (NPAD - N))
    else:
        pad = (k * 41) % N
    return jnp.concatenate([per, pad], axis=1).reshape(NW, NBLK, KB, C)


@jax.jit
def kernel(x, edge_indices, W1, b1, W2, b2, W3, b3, W4, b4):
    src1 = _pad_edges(edge_indices[0, 0], is_dst=False)
    dst1 = _pad_edges(edge_indices[0, 1], is_dst=True)
    src2 = _pad_edges(edge_indices[1, 0], is_dst=False)
    dst2 = _pad_edges(edge_indices[1, 1], is_dst=True)
    dstd = jnp.stack([edge_indices[0, 1].reshape(NS, DNCH, DC),
                      edge_indices[1, 1].reshape(NS, DNCH, DC)])

    _edge_pass, _deg_pass = _sc_kernels()
    dp = _deg_pass(dstd)

    y1 = _prep(x, W1, dp, 0)
    p1 = _edge_pass(y1, src1, dst1)
    y2 = _mid(p1, y1, dp, 0, b1, W2, 1, keep_act=False)
    p2 = _edge_pass(y2, src2, dst2)
    z, y3 = _mid(p2, y2, dp, 1, b2, W3, 0, keep_act=True)
    p3 = _edge_pass(y3, src1, dst1)
    y4 = _mid(p3, y3, dp, 0, b3, W4, 1, keep_act=False)
    p4 = _edge_pass(y4, src2, dst2)
    x_rec = _fin(p4, y4, dp, 1, b4)
    return (z, x_rec)
